# baseline (speedup 1.0000x reference)
import functools

import jax
import jax.numpy as jnp
from jax import lax
from jax.experimental import pallas as pl
from jax.experimental.pallas import tpu as pltpu
from jax.experimental.pallas import tpu_sc as plsc

B = 16384
E = 32
NV = 1_000_000
_L = 16
_VB = 1024
_RNG = 32768
_NBLK = _RNG // _VB
_RING = 8
_DUMP = B
_OUTR = B + _L


def _make_sc_gather():
    info = plsc.get_sparse_core_info()
    NC, NS = info.num_cores, info.num_subcores
    mesh = plsc.VectorSubcoreMesh(core_axis_name="c", subcore_axis_name="s")

    @functools.partial(
        pl.kernel,
        mesh=mesh,
        out_type=[
            jax.ShapeDtypeStruct((_OUTR, 128), jnp.float32),
            jax.ShapeDtypeStruct((_OUTR, 128), jnp.float32),
        ],
        scratch_types=[
            pltpu.VMEM((B,), jnp.int32),
            pltpu.VMEM((B + _L,), jnp.int32),
            pltpu.VMEM((B + _L,), jnp.int32),
            pltpu.VMEM((E, _VB), jnp.float32),
            pltpu.VMEM((E, 64), jnp.float32),
            pltpu.VMEM((_RING * _L, 128), jnp.float32),
            pltpu.VMEM((_RING, _L), jnp.int32),
            pltpu.SemaphoreType.DMA,
            pltpu.SemaphoreType.DMA,
        ],
        compiler_params=pltpu.CompilerParams(needs_layout_passes=False),
    )
    def sc_gather(user_tT, item_tT, tail_u, tail_i, user_ids, item_ids,
                  uout, iout, ids_v, mid_v, mpos_v, blkbuf, tailv,
                  grows, gpos, bsem, ssem):
        wid = lax.axis_index("s") * NC + lax.axis_index("c")
        lo = wid * _RNG
        lane = lax.iota(jnp.int32, _L)

        def one_table(tT, tail_t, ids_hbm, out):
            pltpu.sync_copy(ids_hbm, ids_v)

            def grp(g, off):
                v = ids_v[pl.ds(g * _L, _L)]
                m = jnp.logical_and(v >= lo, v < lo + _RNG)
                s = plsc.cumsum(m.astype(jnp.int32))
                idxs = off + s - 1
                plsc.store_scatter(mid_v, [idxs], v, mask=m)
                plsc.store_scatter(mpos_v, [idxs], g * _L + lane, mask=m)
                return off + plsc.all_reduce_population_count(m)[0]

            n = lax.fori_loop(0, B // _L, grp, 0)
            gmax = (n + _L - 1) // _L

            def match_pass(buf, v0, mlo, mhi, ring0):
                def fgrp(q, ring):
                    mv = mid_v[pl.ds(q * _L, _L)]
                    pv = mpos_v[pl.ds(q * _L, _L)]
                    live = (q * _L + lane) < n
                    bm = jnp.logical_and(
                        live, jnp.logical_and(mv >= mlo, mv < mhi))
                    cnt = plsc.all_reduce_population_count(bm)[0]

                    def emit(r):
                        @pl.when(r >= _RING)
                        def _lagdrain():
                            pltpu.make_async_copy(
                                grows.at[pl.ds(0, _L)],
                                out.at[gpos.at[0]], ssem).wait()

                        bmi = bm.astype(jnp.int32)
                        s = plsc.cumsum(bmi)
                        slot = lax.rem(r, _RING)
                        rbase = slot * _L
                        plsc.store_scatter(
                            gpos, [jnp.full((_L,), slot, jnp.int32), lane],
                            jnp.full((_L,), _DUMP, jnp.int32))
                        plsc.store_scatter(
                            gpos, [jnp.full((_L,), slot, jnp.int32), s - 1],
                            pv, mask=bm)
                        cidv = mv - v0
                        kv = rbase + s - 1
                        for d in range(E):
                            dv = jnp.full((_L,), d, jnp.int32)
                            vals = plsc.load_gather(buf, [dv, cidv], mask=bm)
                            plsc.store_scatter(grows, [kv, dv], vals,
                                               mask=bm)

                        pltpu.async_copy(grows.at[pl.ds(rbase, _L)],
                                         out.at[gpos.at[slot]], ssem)
                        return r + 1

                    return lax.cond(cnt > 0, emit, lambda r: r, ring)

                return lax.fori_loop(0, gmax, fgrp, ring0)

            def blk(b, ring):
                v0 = lo + b * _VB

                def full(r):
                    pltpu.async_copy(tT.at[:, pl.ds(v0, _VB)], blkbuf,
                                     bsem).wait()
                    return match_pass(blkbuf, v0, v0, v0 + _VB, r)

                return lax.cond(v0 + _VB <= NV, full, lambda r: r, ring)

            ring = lax.fori_loop(0, _NBLK, blk, 0)

            def tail(r):
                pltpu.async_copy(tT.at[:, pl.ds(999424, 512)],
                                 blkbuf.at[:, pl.ds(0, 512)], bsem).wait()
                r = match_pass(blkbuf, 999424, 999424, 999936, r)
                pltpu.async_copy(tail_t, tailv, bsem).wait()
                r = match_pass(tailv, 999936, 999936, NV, r)
                return r

            ring = lax.cond(jnp.logical_and(lo <= 999424, 999424 < lo + _RNG),
                            tail, lambda r: r, ring)

            def dr(i, carry):
                @pl.when(i < ring)
                def _drain():
                    pltpu.make_async_copy(grows.at[pl.ds(0, _L)],
                                          out.at[gpos.at[0]], ssem).wait()
                return carry

            lax.fori_loop(0, _RING, dr, 0)

        one_table(user_tT, tail_u, user_ids, uout)
        one_table(item_tT, tail_i, item_ids, iout)

    return sc_gather


_sc_gather = _make_sc_gather()


_BB = 2048


def _mlp_body(ue4, ie4, f, wws, w0u, w0i, w0f, b0, w1, b1, w2, b2,
              wod, cb, out):
    ue = ue4[:, :E]
    ie = ie4[:, :E]
    fv = f[...]
    h = (ue @ w0u[...] + ie @ w0i[...] + fv @ w0f[...] + b0[...])
    h = jnp.maximum(h, 0.0)
    h = jnp.maximum(h @ w1[...] + b1[...], 0.0)
    h = jnp.maximum(h @ w2[...] + b2[...], 0.0)
    logit = fv @ wws[...] + h @ wod[...] + cb[...]
    out[...] = jax.nn.sigmoid(logit)


def _mlp(ue4, ie4, features, wws, w0u, w0i, w0f, b0, w1, b1, w2, b2, wod, cb):
    n_f = features.shape[1]
    d0, d1, d2 = w0u.shape[1], w1.shape[1], w2.shape[1]
    grid = (B // _BB,)
    row = lambda i: (i, 0)
    zero = lambda i: (0, 0)
    return pl.pallas_call(
        _mlp_body,
        grid=grid,
        in_specs=[
            pl.BlockSpec((_BB, 128), row),
            pl.BlockSpec((_BB, 128), row),
            pl.BlockSpec((_BB, n_f), row),
            pl.BlockSpec((n_f, 1), zero),
            pl.BlockSpec((E, d0), zero),
            pl.BlockSpec((E, d0), zero),
            pl.BlockSpec((n_f, d0), zero),
            pl.BlockSpec((1, d0), zero),
            pl.BlockSpec((d0, d1), zero),
            pl.BlockSpec((1, d1), zero),
            pl.BlockSpec((d1, d2), zero),
            pl.BlockSpec((1, d2), zero),
            pl.BlockSpec((d2, 1), zero),
            pl.BlockSpec((1, 1), zero),
        ],
        out_specs=pl.BlockSpec((_BB, 1), row),
        out_shape=jax.ShapeDtypeStruct((B, 1), jnp.float32),
        compiler_params=pltpu.CompilerParams(
            dimension_semantics=("arbitrary",),
        ),
    )(ue4, ie4, features, wws, w0u, w0i, w0f, b0, w1, b1, w2, b2, wod, cb)


def kernel(user_ids, item_ids, features, user_table, item_table,
           W_wide, b_wide, W0, b0, W1, b1, W2, b2, Wo, bo):
    ue4, ie4 = _sc_gather(user_table.T, item_table.T,
                          user_table[999936:, :].T, item_table[999936:, :].T,
                          user_ids, item_ids)

    wo0 = Wo[0, 0]
    wws = W_wide * wo0
    wod = Wo[1:, :]
    cb = (b_wide * wo0 + bo).reshape(1, 1)
    w0u = W0[:E, :]
    w0i = W0[E:2 * E, :]
    w0f = W0[2 * E:, :]

    return _mlp(ue4, ie4, features,
                wws, w0u, w0i, w0f, b0.reshape(1, -1),
                W1, b1.reshape(1, -1), W2, b2.reshape(1, -1), wod, cb)


# speedup vs baseline: 1.0014x; 1.0014x over previous
import functools

import jax
import jax.numpy as jnp
from jax import lax
from jax.experimental import pallas as pl
from jax.experimental.pallas import tpu as pltpu
from jax.experimental.pallas import tpu_sc as plsc

B = 16384
E = 32
NV = 1_000_000
_L = 16
_VB = 1024
_RNG = 32768
_NBLK = _RNG // _VB
_RING = 8
_DUMP = B
_OUTR = B + _L


def _make_sc_gather():
    info = plsc.get_sparse_core_info()
    NC, NS = info.num_cores, info.num_subcores
    mesh = plsc.VectorSubcoreMesh(core_axis_name="c", subcore_axis_name="s")

    @functools.partial(
        pl.kernel,
        mesh=mesh,
        out_type=[
            jax.ShapeDtypeStruct((_OUTR, 128), jnp.float32),
            jax.ShapeDtypeStruct((_OUTR, 128), jnp.float32),
        ],
        scratch_types=[
            pltpu.VMEM((B,), jnp.int32),
            pltpu.VMEM((B + _L,), jnp.int32),
            pltpu.VMEM((B + _L,), jnp.int32),
            pltpu.VMEM((E, _VB), jnp.float32),
            pltpu.VMEM((E, 64), jnp.float32),
            pltpu.VMEM((_RING * _L, 128), jnp.float32),
            pltpu.VMEM((_RING, _L), jnp.int32),
            pltpu.SemaphoreType.DMA,
            pltpu.SemaphoreType.DMA,
        ],
        compiler_params=pltpu.CompilerParams(needs_layout_passes=False),
    )
    def sc_gather(user_tT, item_tT, tail_u, tail_i, user_ids, item_ids,
                  uout, iout, ids_v, mid_v, mpos_v, blkbuf, tailv,
                  grows, gpos, bsem, ssem):
        wid = lax.axis_index("s") * NC + lax.axis_index("c")
        lo = wid * _RNG
        lane = lax.iota(jnp.int32, _L)

        def one_table(tT, tail_t, ids_hbm, out):
            pltpu.sync_copy(ids_hbm, ids_v)

            def grp(g, off):
                v = plsc.load_gather(ids_v, [g * _L + lane])
                m = jnp.logical_and(v >= lo, v < lo + _RNG)
                s = plsc.cumsum(m.astype(jnp.int32))
                idxs = off + s - 1
                plsc.store_scatter(mid_v, [idxs], v, mask=m)
                plsc.store_scatter(mpos_v, [idxs], g * _L + lane, mask=m)
                return off + plsc.all_reduce_population_count(m)[0]

            n = lax.fori_loop(0, B // _L, grp, 0)
            gmax = (n + _L - 1) // _L

            def match_pass(buf, v0, mlo, mhi, ring0):
                def fgrp(q, ring):
                    mv = plsc.load_gather(mid_v, [q * _L + lane])
                    pv = plsc.load_gather(mpos_v, [q * _L + lane])
                    live = (q * _L + lane) < n
                    bm = jnp.logical_and(
                        live, jnp.logical_and(mv >= mlo, mv < mhi))
                    cnt = plsc.all_reduce_population_count(bm)[0]

                    def emit(r):
                        @pl.when(r >= _RING)
                        def _lagdrain():
                            pltpu.make_async_copy(
                                grows.at[pl.ds(0, _L)],
                                out.at[gpos.at[0]], ssem).wait()

                        bmi = bm.astype(jnp.int32)
                        s = plsc.cumsum(bmi)
                        slot = lax.rem(r, _RING)
                        rbase = slot * _L
                        plsc.store_scatter(
                            gpos, [jnp.full((_L,), slot, jnp.int32), lane],
                            jnp.full((_L,), _DUMP, jnp.int32))
                        plsc.store_scatter(
                            gpos, [jnp.full((_L,), slot, jnp.int32), s - 1],
                            pv, mask=bm)
                        cidv = mv - v0
                        kv = rbase + s - 1
                        for d in range(E):
                            dv = jnp.full((_L,), d, jnp.int32)
                            vals = plsc.load_gather(buf, [dv, cidv], mask=bm)
                            plsc.store_scatter(grows, [kv, dv], vals,
                                               mask=bm)

                        pltpu.async_copy(grows.at[pl.ds(rbase, _L)],
                                         out.at[gpos.at[slot]], ssem)
                        return r + 1

                    return lax.cond(cnt > 0, emit, lambda r: r, ring)

                return lax.fori_loop(0, gmax, fgrp, ring0)

            def blk(b, ring):
                v0 = lo + b * _VB

                def full(r):
                    pltpu.async_copy(tT.at[:, pl.ds(v0, _VB)], blkbuf,
                                     bsem).wait()
                    return match_pass(blkbuf, v0, v0, v0 + _VB, r)

                return lax.cond(v0 + _VB <= NV, full, lambda r: r, ring)

            ring = lax.fori_loop(0, _NBLK, blk, 0)

            def tail(r):
                pltpu.async_copy(tT.at[:, pl.ds(999424, 512)],
                                 blkbuf.at[:, pl.ds(0, 512)], bsem).wait()
                r = match_pass(blkbuf, 999424, 999424, 999936, r)
                pltpu.async_copy(tail_t, tailv, bsem).wait()
                r = match_pass(tailv, 999936, 999936, NV, r)
                return r

            ring = lax.cond(jnp.logical_and(lo <= 999424, 999424 < lo + _RNG),
                            tail, lambda r: r, ring)

            def dr(i, carry):
                @pl.when(i < ring)
                def _drain():
                    pltpu.make_async_copy(grows.at[pl.ds(0, _L)],
                                          out.at[gpos.at[0]], ssem).wait()
                return carry

            lax.fori_loop(0, _RING, dr, 0)

        one_table(user_tT, tail_u, user_ids, uout)
        one_table(item_tT, tail_i, item_ids, iout)

    return sc_gather


_sc_gather = _make_sc_gather()


_BB = 2048


def _mlp_body(ue4, ie4, f, wws, w0u, w0i, w0f, b0, w1, b1, w2, b2,
              wod, cb, out):
    ue = ue4[:, :E]
    ie = ie4[:, :E]
    fv = f[...]
    h = (ue @ w0u[...] + ie @ w0i[...] + fv @ w0f[...] + b0[...])
    h = jnp.maximum(h, 0.0)
    h = jnp.maximum(h @ w1[...] + b1[...], 0.0)
    h = jnp.maximum(h @ w2[...] + b2[...], 0.0)
    logit = fv @ wws[...] + h @ wod[...] + cb[...]
    out[...] = jax.nn.sigmoid(logit)


def _mlp(ue4, ie4, features, wws, w0u, w0i, w0f, b0, w1, b1, w2, b2, wod, cb):
    n_f = features.shape[1]
    d0, d1, d2 = w0u.shape[1], w1.shape[1], w2.shape[1]
    grid = (B // _BB,)
    row = lambda i: (i, 0)
    zero = lambda i: (0, 0)
    return pl.pallas_call(
        _mlp_body,
        grid=grid,
        in_specs=[
            pl.BlockSpec((_BB, 128), row),
            pl.BlockSpec((_BB, 128), row),
            pl.BlockSpec((_BB, n_f), row),
            pl.BlockSpec((n_f, 1), zero),
            pl.BlockSpec((E, d0), zero),
            pl.BlockSpec((E, d0), zero),
            pl.BlockSpec((n_f, d0), zero),
            pl.BlockSpec((1, d0), zero),
            pl.BlockSpec((d0, d1), zero),
            pl.BlockSpec((1, d1), zero),
            pl.BlockSpec((d1, d2), zero),
            pl.BlockSpec((1, d2), zero),
            pl.BlockSpec((d2, 1), zero),
            pl.BlockSpec((1, 1), zero),
        ],
        out_specs=pl.BlockSpec((_BB, 1), row),
        out_shape=jax.ShapeDtypeStruct((B, 1), jnp.float32),
        compiler_params=pltpu.CompilerParams(
            dimension_semantics=("arbitrary",),
        ),
    )(ue4, ie4, features, wws, w0u, w0i, w0f, b0, w1, b1, w2, b2, wod, cb)


def kernel(user_ids, item_ids, features, user_table, item_table,
           W_wide, b_wide, W0, b0, W1, b1, W2, b2, Wo, bo):
    ue4, ie4 = _sc_gather(user_table.T, item_table.T,
                          user_table[999936:, :].T, item_table[999936:, :].T,
                          user_ids, item_ids)

    wo0 = Wo[0, 0]
    wws = W_wide * wo0
    wod = Wo[1:, :]
    cb = (b_wide * wo0 + bo).reshape(1, 1)
    w0u = W0[:E, :]
    w0i = W0[E:2 * E, :]
    w0f = W0[2 * E:, :]

    return _mlp(ue4, ie4, features,
                wws, w0u, w0i, w0f, b0.reshape(1, -1),
                W1, b1.reshape(1, -1), W2, b2.reshape(1, -1), wod, cb)


# speedup vs baseline: 27.9850x; 27.9472x over previous
import functools

import jax
import jax.numpy as jnp
from jax import lax
from jax.experimental import pallas as pl
from jax.experimental.pallas import tpu as pltpu
from jax.experimental.pallas import tpu_sc as plsc

B = 16384
E = 32
NV = 1_000_000
_L = 16
_VB = 1024
_RNG = 32768
_NBLK = _RNG // _VB
_RING = 8
_NW = 32
_OUTR = B + _NW * _L


def _make_sc_gather():
    info = plsc.get_sparse_core_info()
    NC, NS = info.num_cores, info.num_subcores
    mesh = plsc.VectorSubcoreMesh(core_axis_name="c", subcore_axis_name="s")

    @functools.partial(
        pl.kernel,
        mesh=mesh,
        out_type=[
            jax.ShapeDtypeStruct((_OUTR, 128), jnp.float32),
            jax.ShapeDtypeStruct((_OUTR, 128), jnp.float32),
        ],
        scratch_types=[
            pltpu.VMEM((B,), jnp.int32),
            pltpu.VMEM((B + _L,), jnp.int32),
            pltpu.VMEM((B + _L,), jnp.int32),
            pltpu.VMEM((E, _VB), jnp.float32),
            pltpu.VMEM((E, 64), jnp.float32),
            pltpu.VMEM((_RING * _L, 128), jnp.float32),
            pltpu.VMEM((_RING, _L), jnp.int32),
            pltpu.SemaphoreType.DMA,
            pltpu.SemaphoreType.DMA,
        ],
        compiler_params=pltpu.CompilerParams(needs_layout_passes=False),
    )
    def sc_gather(user_tT, item_tT, tail_u, tail_i, user_ids, item_ids,
                  uout, iout, ids_v, mid_v, mpos_v, blkbuf, tailv,
                  grows, gpos, bsem, ssem):
        wid = lax.axis_index("s") * NC + lax.axis_index("c")
        lo = wid * _RNG
        lane = lax.iota(jnp.int32, _L)

        def one_table(tT, tail_t, ids_hbm, out):
            pltpu.sync_copy(ids_hbm, ids_v)

            def grp(g, off):
                v = plsc.load_gather(ids_v, [g * _L + lane])
                m = jnp.logical_and(v >= lo, v < lo + _RNG)
                s = plsc.cumsum(m.astype(jnp.int32))
                idxs = off + s - 1
                plsc.store_scatter(mid_v, [idxs], v, mask=m)
                plsc.store_scatter(mpos_v, [idxs], g * _L + lane, mask=m)
                return off + plsc.all_reduce_population_count(m)[0]

            n = lax.fori_loop(0, B // _L, grp, 0)
            gmax = (n + _L - 1) // _L

            def match_pass(buf, v0, mlo, mhi, ring0):
                def fgrp(q, ring):
                    mv = plsc.load_gather(mid_v, [q * _L + lane])
                    pv = plsc.load_gather(mpos_v, [q * _L + lane])
                    live = (q * _L + lane) < n
                    bm = jnp.logical_and(
                        live, jnp.logical_and(mv >= mlo, mv < mhi))
                    cnt = plsc.all_reduce_population_count(bm)[0]

                    def emit(r):
                        @pl.when(r >= _RING)
                        def _lagdrain():
                            pltpu.make_async_copy(
                                grows.at[pl.ds(0, _L)],
                                out.at[gpos.at[0]], ssem).wait()

                        bmi = bm.astype(jnp.int32)
                        s = plsc.cumsum(bmi)
                        slot = lax.rem(r, _RING)
                        rbase = slot * _L
                        plsc.store_scatter(
                            gpos, [jnp.full((_L,), slot, jnp.int32), lane],
                            B + wid * _L + lane)
                        plsc.store_scatter(
                            gpos, [jnp.full((_L,), slot, jnp.int32), s - 1],
                            pv, mask=bm)
                        cidv = mv - v0
                        kv = rbase + s - 1
                        for d in range(E):
                            dv = jnp.full((_L,), d, jnp.int32)
                            vals = plsc.load_gather(buf, [dv, cidv], mask=bm)
                            plsc.store_scatter(grows, [kv, dv], vals,
                                               mask=bm)

                        pltpu.async_copy(grows.at[pl.ds(rbase, _L)],
                                         out.at[gpos.at[slot]], ssem)
                        return r + 1

                    return lax.cond(cnt > 0, emit, lambda r: r, ring)

                return lax.fori_loop(0, gmax, fgrp, ring0)

            def blk(b, ring):
                v0 = lo + b * _VB

                def full(r):
                    pltpu.async_copy(tT.at[:, pl.ds(v0, _VB)], blkbuf,
                                     bsem).wait()
                    return match_pass(blkbuf, v0, v0, v0 + _VB, r)

                return lax.cond(v0 + _VB <= NV, full, lambda r: r, ring)

            ring = lax.fori_loop(0, _NBLK, blk, 0)

            def tail(r):
                pltpu.async_copy(tT.at[:, pl.ds(999424, 512)],
                                 blkbuf.at[:, pl.ds(0, 512)], bsem).wait()
                r = match_pass(blkbuf, 999424, 999424, 999936, r)
                pltpu.async_copy(tail_t, tailv, bsem).wait()
                r = match_pass(tailv, 999936, 999936, NV, r)
                return r

            ring = lax.cond(jnp.logical_and(lo <= 999424, 999424 < lo + _RNG),
                            tail, lambda r: r, ring)

            def dr(i, carry):
                @pl.when(i < ring)
                def _drain():
                    pltpu.make_async_copy(grows.at[pl.ds(0, _L)],
                                          out.at[gpos.at[0]], ssem).wait()
                return carry

            lax.fori_loop(0, _RING, dr, 0)

        one_table(user_tT, tail_u, user_ids, uout)
        one_table(item_tT, tail_i, item_ids, iout)

    return sc_gather


_sc_gather = _make_sc_gather()


_BB = 2048


def _mlp_body(ue4, ie4, f, wws, w0u, w0i, w0f, b0, w1, b1, w2, b2,
              wod, cb, out):
    ue = ue4[:, :E]
    ie = ie4[:, :E]
    fv = f[...]
    h = (ue @ w0u[...] + ie @ w0i[...] + fv @ w0f[...] + b0[...])
    h = jnp.maximum(h, 0.0)
    h = jnp.maximum(h @ w1[...] + b1[...], 0.0)
    h = jnp.maximum(h @ w2[...] + b2[...], 0.0)
    logit = fv @ wws[...] + h @ wod[...] + cb[...]
    out[...] = jax.nn.sigmoid(logit)


def _mlp(ue4, ie4, features, wws, w0u, w0i, w0f, b0, w1, b1, w2, b2, wod, cb):
    n_f = features.shape[1]
    d0, d1, d2 = w0u.shape[1], w1.shape[1], w2.shape[1]
    grid = (B // _BB,)
    row = lambda i: (i, 0)
    zero = lambda i: (0, 0)
    return pl.pallas_call(
        _mlp_body,
        grid=grid,
        in_specs=[
            pl.BlockSpec((_BB, 128), row),
            pl.BlockSpec((_BB, 128), row),
            pl.BlockSpec((_BB, n_f), row),
            pl.BlockSpec((n_f, 1), zero),
            pl.BlockSpec((E, d0), zero),
            pl.BlockSpec((E, d0), zero),
            pl.BlockSpec((n_f, d0), zero),
            pl.BlockSpec((1, d0), zero),
            pl.BlockSpec((d0, d1), zero),
            pl.BlockSpec((1, d1), zero),
            pl.BlockSpec((d1, d2), zero),
            pl.BlockSpec((1, d2), zero),
            pl.BlockSpec((d2, 1), zero),
            pl.BlockSpec((1, 1), zero),
        ],
        out_specs=pl.BlockSpec((_BB, 1), row),
        out_shape=jax.ShapeDtypeStruct((B, 1), jnp.float32),
        compiler_params=pltpu.CompilerParams(
            dimension_semantics=("arbitrary",),
        ),
    )(ue4, ie4, features, wws, w0u, w0i, w0f, b0, w1, b1, w2, b2, wod, cb)


def kernel(user_ids, item_ids, features, user_table, item_table,
           W_wide, b_wide, W0, b0, W1, b1, W2, b2, Wo, bo):
    ue4, ie4 = _sc_gather(user_table.T, item_table.T,
                          user_table[999936:, :].T, item_table[999936:, :].T,
                          user_ids, item_ids)

    wo0 = Wo[0, 0]
    wws = W_wide * wo0
    wod = Wo[1:, :]
    cb = (b_wide * wo0 + bo).reshape(1, 1)
    w0u = W0[:E, :]
    w0i = W0[E:2 * E, :]
    w0f = W0[2 * E:, :]

    return _mlp(ue4, ie4, features,
                wws, w0u, w0i, w0f, b0.reshape(1, -1),
                W1, b1.reshape(1, -1), W2, b2.reshape(1, -1), wod, cb)


# speedup vs baseline: 28.9451x; 1.0343x over previous
import functools

import jax
import jax.numpy as jnp
from jax import lax
from jax.experimental import pallas as pl
from jax.experimental.pallas import tpu as pltpu
from jax.experimental.pallas import tpu_sc as plsc

B = 16384
E = 32
NV = 1_000_000
_L = 16
_VB = 512
_RNG = 32768
_NBLK = _RNG // _VB
_RING = 8
_NW = 32
_OUTR = B + _NW * _L


def _make_sc_gather():
    info = plsc.get_sparse_core_info()
    NC, NS = info.num_cores, info.num_subcores
    mesh = plsc.VectorSubcoreMesh(core_axis_name="c", subcore_axis_name="s")

    @functools.partial(
        pl.kernel,
        mesh=mesh,
        out_type=[
            jax.ShapeDtypeStruct((_OUTR, 128), jnp.float32),
            jax.ShapeDtypeStruct((_OUTR, 128), jnp.float32),
        ],
        scratch_types=[
            pltpu.VMEM((B,), jnp.int32),
            pltpu.VMEM((B + _L,), jnp.int32),
            pltpu.VMEM((B + _L,), jnp.int32),
            pltpu.VMEM((E, _VB), jnp.float32),
            pltpu.VMEM((E, _VB), jnp.float32),
            pltpu.VMEM((E, 64), jnp.float32),
            pltpu.VMEM((_RING * _L, 128), jnp.float32),
            pltpu.VMEM((_RING, _L), jnp.int32),
            pltpu.SemaphoreType.DMA,
            pltpu.SemaphoreType.DMA,
            pltpu.SemaphoreType.DMA,
        ],
        compiler_params=pltpu.CompilerParams(needs_layout_passes=False),
    )
    def sc_gather(user_tT, item_tT, tail_u, tail_i, user_ids, item_ids,
                  uout, iout, ids_v, mid_v, mpos_v, blkbuf, blkbuf2, tailv,
                  grows, gpos, bsem, bsem2, ssem):
        wid = lax.axis_index("s") * NC + lax.axis_index("c")
        lo = wid * _RNG
        lane = lax.iota(jnp.int32, _L)

        def one_table(tT, tail_t, ids_hbm, out):
            pltpu.sync_copy(ids_hbm, ids_v)

            def grp(g, off):
                v = plsc.load_gather(ids_v, [g * _L + lane])
                m = jnp.logical_and(v >= lo, v < lo + _RNG)
                s = plsc.cumsum(m.astype(jnp.int32))
                idxs = off + s - 1
                plsc.store_scatter(mid_v, [idxs], v, mask=m)
                plsc.store_scatter(mpos_v, [idxs], g * _L + lane, mask=m)
                return off + plsc.all_reduce_population_count(m)[0]

            n = lax.fori_loop(0, B // _L, grp, 0)
            gmax = (n + _L - 1) // _L

            def match_pass(buf, v0, mlo, mhi, ring0):
                def fgrp(q, ring):
                    mv = plsc.load_gather(mid_v, [q * _L + lane])
                    pv = plsc.load_gather(mpos_v, [q * _L + lane])
                    live = (q * _L + lane) < n
                    bm = jnp.logical_and(
                        live, jnp.logical_and(mv >= mlo, mv < mhi))
                    cnt = plsc.all_reduce_population_count(bm)[0]

                    def emit(r):
                        @pl.when(r >= _RING)
                        def _lagdrain():
                            pltpu.make_async_copy(
                                grows.at[pl.ds(0, _L)],
                                out.at[gpos.at[0]], ssem).wait()

                        bmi = bm.astype(jnp.int32)
                        s = plsc.cumsum(bmi)
                        slot = lax.rem(r, _RING)
                        rbase = slot * _L
                        plsc.store_scatter(
                            gpos, [jnp.full((_L,), slot, jnp.int32), lane],
                            B + wid * _L + lane)
                        plsc.store_scatter(
                            gpos, [jnp.full((_L,), slot, jnp.int32), s - 1],
                            pv, mask=bm)
                        cidv = mv - v0
                        kv = rbase + s - 1
                        for d in range(E):
                            dv = jnp.full((_L,), d, jnp.int32)
                            vals = plsc.load_gather(buf, [dv, cidv], mask=bm)
                            plsc.store_scatter(grows, [kv, dv], vals,
                                               mask=bm)

                        pltpu.async_copy(grows.at[pl.ds(rbase, _L)],
                                         out.at[gpos.at[slot]], ssem)
                        return r + 1

                    return lax.cond(cnt > 0, emit, lambda r: r, ring)

                return lax.fori_loop(0, gmax, fgrp, ring0)

            def issue_blk(b, buf, sem):
                v0 = lo + b * _VB

                @pl.when(jnp.logical_and(b < _NBLK, v0 + _VB <= NV))
                def _pref():
                    pltpu.async_copy(tT.at[:, pl.ds(v0, _VB)], buf, sem)

            def consume_blk(b, buf, sem, r):
                v0 = lo + b * _VB

                def go(rr):
                    pltpu.make_async_copy(tT.at[:, pl.ds(lo, _VB)], buf,
                                          sem).wait()
                    return match_pass(buf, v0, v0, v0 + _VB, rr)

                return lax.cond(v0 + _VB <= NV, go, lambda rr: rr, r)

            issue_blk(0, blkbuf, bsem)

            def pair(p, ring):
                b0 = 2 * p
                issue_blk(b0 + 1, blkbuf2, bsem2)
                ring = consume_blk(b0, blkbuf, bsem, ring)
                issue_blk(b0 + 2, blkbuf, bsem)
                ring = consume_blk(b0 + 1, blkbuf2, bsem2, ring)
                return ring

            ring = lax.fori_loop(0, _NBLK // 2, pair, 0)

            def tail(r):
                pltpu.async_copy(tT.at[:, pl.ds(999424, 512)],
                                 blkbuf.at[:, pl.ds(0, 512)], bsem).wait()
                r = match_pass(blkbuf, 999424, 999424, 999936, r)
                pltpu.async_copy(tail_t, tailv, bsem).wait()
                r = match_pass(tailv, 999936, 999936, NV, r)
                return r

            ring = lax.cond(jnp.logical_and(lo <= 999424, 999424 < lo + _RNG),
                            tail, lambda r: r, ring)

            def dr(i, carry):
                @pl.when(i < ring)
                def _drain():
                    pltpu.make_async_copy(grows.at[pl.ds(0, _L)],
                                          out.at[gpos.at[0]], ssem).wait()
                return carry

            lax.fori_loop(0, _RING, dr, 0)

        one_table(user_tT, tail_u, user_ids, uout)
        one_table(item_tT, tail_i, item_ids, iout)

    return sc_gather


_sc_gather = _make_sc_gather()


_BB = 2048


def _mlp_body(ue4, ie4, f, wws, w0u, w0i, w0f, b0, w1, b1, w2, b2,
              wod, cb, out):
    ue = ue4[:, :E]
    ie = ie4[:, :E]
    fv = f[...]
    h = (ue @ w0u[...] + ie @ w0i[...] + fv @ w0f[...] + b0[...])
    h = jnp.maximum(h, 0.0)
    h = jnp.maximum(h @ w1[...] + b1[...], 0.0)
    h = jnp.maximum(h @ w2[...] + b2[...], 0.0)
    logit = fv @ wws[...] + h @ wod[...] + cb[...]
    out[...] = jax.nn.sigmoid(logit)


def _mlp(ue4, ie4, features, wws, w0u, w0i, w0f, b0, w1, b1, w2, b2, wod, cb):
    n_f = features.shape[1]
    d0, d1, d2 = w0u.shape[1], w1.shape[1], w2.shape[1]
    grid = (B // _BB,)
    row = lambda i: (i, 0)
    zero = lambda i: (0, 0)
    return pl.pallas_call(
        _mlp_body,
        grid=grid,
        in_specs=[
            pl.BlockSpec((_BB, 128), row),
            pl.BlockSpec((_BB, 128), row),
            pl.BlockSpec((_BB, n_f), row),
            pl.BlockSpec((n_f, 1), zero),
            pl.BlockSpec((E, d0), zero),
            pl.BlockSpec((E, d0), zero),
            pl.BlockSpec((n_f, d0), zero),
            pl.BlockSpec((1, d0), zero),
            pl.BlockSpec((d0, d1), zero),
            pl.BlockSpec((1, d1), zero),
            pl.BlockSpec((d1, d2), zero),
            pl.BlockSpec((1, d2), zero),
            pl.BlockSpec((d2, 1), zero),
            pl.BlockSpec((1, 1), zero),
        ],
        out_specs=pl.BlockSpec((_BB, 1), row),
        out_shape=jax.ShapeDtypeStruct((B, 1), jnp.float32),
        compiler_params=pltpu.CompilerParams(
            dimension_semantics=("arbitrary",),
        ),
    )(ue4, ie4, features, wws, w0u, w0i, w0f, b0, w1, b1, w2, b2, wod, cb)


def kernel(user_ids, item_ids, features, user_table, item_table,
           W_wide, b_wide, W0, b0, W1, b1, W2, b2, Wo, bo):
    ue4, ie4 = _sc_gather(user_table.T, item_table.T,
                          user_table[999936:, :].T, item_table[999936:, :].T,
                          user_ids, item_ids)

    wo0 = Wo[0, 0]
    wws = W_wide * wo0
    wod = Wo[1:, :]
    cb = (b_wide * wo0 + bo).reshape(1, 1)
    w0u = W0[:E, :]
    w0i = W0[E:2 * E, :]
    w0f = W0[2 * E:, :]

    return _mlp(ue4, ie4, features,
                wws, w0u, w0i, w0f, b0.reshape(1, -1),
                W1, b1.reshape(1, -1), W2, b2.reshape(1, -1), wod, cb)
